# baseline (speedup 1.0000x reference)
import functools

import jax
import jax.numpy as jnp
from jax import lax
from jax.experimental import pallas as pl
from jax.experimental.pallas import tpu as pltpu
from jax.experimental.pallas import tpu_sc as plsc

D = 64
L = 200
NC, NS = 2, 16
NW = NC * NS
BATCH = 4096
BB = BATCH // NW
DT, DI = D // 8, 8
LANES = 16
SUP = 2
RB = 2
NS_T = L // SUP
NP = NS_T // RB
PSTRIDE = D + 1


@functools.cache
def _embed_kernel():
    mesh = plsc.VectorSubcoreMesh(core_axis_name="c", subcore_axis_name="s")

    @functools.partial(
        pl.kernel,
        mesh=mesh,
        compiler_params=pltpu.CompilerParams(
            use_tc_tiling_on_sc=False, needs_layout_passes=False
        ),
        out_type=jax.ShapeDtypeStruct((L, DT, NW, DI, BB), jnp.float32),
        scratch_types=[
            pltpu.VMEM((BB // 2 * L,), jnp.int32),
            pltpu.VMEM((L * BB,), jnp.int32),
            pltpu.VMEM((RB, SUP * BB, D), jnp.float32),
            pltpu.VMEM((BB * PSTRIDE,), jnp.float32),
            pltpu.VMEM((RB * SUP, DT, DI, BB), jnp.float32),
            pltpu.VMEM((L, D), jnp.float32),
            pltpu.SemaphoreType.DMA,
            pltpu.SemaphoreType.DMA,
            pltpu.SemaphoreType.DMA,
            pltpu.SemaphoreType.DMA,
            pltpu.SemaphoreType.DMA,
            pltpu.SemaphoreType.DMA,
        ],
    )
    def body(x_hbm, tok_hbm, pos_hbm, out_hbm, x_v, xt_v, rows_v, p_v, ob_v,
             pos_v, g0, g1, w0, w1, w2, w3):
        gsem = [g0, g1]
        wsem = [w0, w1, w2, w3]
        j = lax.axis_index("s") * NC + lax.axis_index("c")
        pltpu.sync_copy(pos_hbm, pos_v)

        lane = lax.iota(jnp.int32, LANES)
        m_idx = [g * LANES + lane for g in range(BB // LANES)]
        m65 = [(g * LANES + lane) * PSTRIDE for g in range(BB // LANES)]
        mL = [(g * LANES + lane) * L for g in range(BB // 2 // LANES)]

        for h in range(2):
            pltpu.sync_copy(
                x_hbm.at[pl.ds((j * BB + h * BB // 2) * L, BB // 2 * L)], x_v
            )

            @plsc.parallel_loop(0, L)
            def xpose_body(l):
                l_splat = jnp.full((LANES,), l, jnp.int32)
                for g in range(BB // 2 // LANES):
                    v = plsc.load_gather(x_v, [mL[g] + l_splat])
                    xt_v[pl.ds(l * BB + h * (BB // 2) + g * LANES, LANES)] = v

        for rb in range(RB):
            pltpu.async_copy(
                tok_hbm.at[xt_v.at[pl.ds(rb * SUP * BB, SUP * BB)]],
                rows_v.at[rb], gsem[rb]
            )

        def compute_tile(l, rb, q, ob):
            pchunk = [pos_v[l, pl.ds(k * LANES, LANES)] for k in range(D // LANES)]

            @plsc.parallel_loop(0, BB // 4, unroll=4)
            def m_body(mq):
                for mi in range(4):
                    m = mq * 4 + mi
                    for k in range(D // LANES):
                        v = rows_v[rb, q * BB + m, pl.ds(k * LANES, LANES)] + pchunk[k]
                        p_v[pl.ds(m * PSTRIDE + k * LANES, LANES)] = v

            @plsc.parallel_loop(0, DT, unroll=2)
            def dt_body(dt):
                for di in range(DI):
                    d_splat = jnp.full((LANES,), dt * DI + di, jnp.int32)
                    for g in range(BB // LANES):
                        v = plsc.load_gather(p_v, [m65[g] + d_splat])
                        ob_v[ob, dt, di, pl.ds(g * LANES, LANES)] = v

        def p_body(p, carry):
            for rb in range(RB):
                s = p * RB + rb
                pltpu.make_async_copy(
                    tok_hbm.at[xt_v.at[pl.ds(s * SUP * BB, SUP * BB)]],
                    rows_v.at[rb], gsem[rb]
                ).wait()

                for q in range(SUP):
                    l = s * SUP + q
                    ob = rb * SUP + q

                    @pl.when(p > 0)
                    def _():
                        pltpu.make_async_copy(
                            ob_v.at[ob], out_hbm.at[l - RB * SUP, :, j], wsem[ob]
                        ).wait()

                    compute_tile(l, rb, q, ob)
                    pltpu.async_copy(ob_v.at[ob], out_hbm.at[l, :, j], wsem[ob])

                @pl.when(p < NP - 1)
                def _():
                    pltpu.async_copy(
                        tok_hbm.at[
                            xt_v.at[pl.ds((s + RB) * SUP * BB, SUP * BB)]
                        ],
                        rows_v.at[rb], gsem[rb]
                    )

            return carry

        lax.fori_loop(0, NP, p_body, 0)
        for ob in range(RB * SUP):
            pltpu.make_async_copy(
                ob_v.at[ob], out_hbm.at[L - RB * SUP + ob, :, j], wsem[ob]
            ).wait()

    return body


def kernel(x, token_table, pos_table):
    xi = x.reshape(-1).astype(jnp.int32)
    buf = _embed_kernel()(xi, token_table, pos_table)
    return buf.transpose(2, 4, 0, 1, 3).reshape(BATCH, L, D)


# speedup vs baseline: 1.5353x; 1.5353x over previous
import functools

import jax
import jax.numpy as jnp
from jax import lax
from jax.experimental import pallas as pl
from jax.experimental.pallas import tpu as pltpu
from jax.experimental.pallas import tpu_sc as plsc

D = 64
L = 200
NC, NS = 2, 16
NW = NC * NS
BATCH = 4096
BB = BATCH // NW
DT, DI = D // 8, 8
LANES = 16
RB = 4
NP = L // RB
PSTRIDE = D + 1


@functools.cache
def _embed_kernel():
    mesh = plsc.VectorSubcoreMesh(core_axis_name="c", subcore_axis_name="s")

    @functools.partial(
        pl.kernel,
        mesh=mesh,
        compiler_params=pltpu.CompilerParams(
            use_tc_tiling_on_sc=False, needs_layout_passes=False
        ),
        out_type=jax.ShapeDtypeStruct((L, DT, NW, DI, BB), jnp.float32),
        scratch_types=[
            pltpu.VMEM((BB // 2 * L,), jnp.int32),
            pltpu.VMEM((L, BB), jnp.int32),
            pltpu.VMEM((RB, BB, D), jnp.float32),
            pltpu.VMEM((BB * PSTRIDE,), jnp.float32),
            pltpu.VMEM((RB, DT, DI, BB), jnp.float32),
            pltpu.VMEM((L, D), jnp.float32),
            pltpu.SemaphoreType.DMA,
            pltpu.SemaphoreType.DMA,
            pltpu.SemaphoreType.DMA,
            pltpu.SemaphoreType.DMA,
            pltpu.SemaphoreType.DMA,
            pltpu.SemaphoreType.DMA,
            pltpu.SemaphoreType.DMA,
            pltpu.SemaphoreType.DMA,
        ],
    )
    def body(x_hbm, tok_hbm, pos_hbm, out_hbm, x_v, xt_v, rows_v, p_v, ob_v,
             pos_v, g0, g1, g2, g3, w0, w1, w2, w3):
        gsem = [g0, g1, g2, g3]
        wsem = [w0, w1, w2, w3]
        j = lax.axis_index("s") * NC + lax.axis_index("c")
        pltpu.sync_copy(pos_hbm, pos_v)

        lane = lax.iota(jnp.int32, LANES)
        m65 = [(g * LANES + lane) * PSTRIDE for g in range(BB // LANES)]
        mL = [(g * LANES + lane) * L for g in range(BB // 2 // LANES)]

        for h in range(2):
            pltpu.sync_copy(
                x_hbm.at[pl.ds((j * BB + h * BB // 2) * L, BB // 2 * L)], x_v
            )

            @plsc.parallel_loop(0, L)
            def xpose_body(l):
                l_splat = jnp.full((LANES,), l, jnp.int32)
                for g in range(BB // 2 // LANES):
                    v = plsc.load_gather(x_v, [mL[g] + l_splat])
                    xt_v[l, pl.ds(h * (BB // 2) + g * LANES, LANES)] = v

        for rb in range(RB):
            pltpu.async_copy(tok_hbm.at[xt_v.at[rb]], rows_v.at[rb], gsem[rb])

        def compute_tile(l, rb):
            pchunk = [pos_v[l, pl.ds(k * LANES, LANES)] for k in range(D // LANES)]

            @plsc.parallel_loop(0, BB // 4, unroll=2)
            def m_body(mq):
                for mi in range(4):
                    m = mq * 4 + mi
                    for k in range(D // LANES):
                        v = rows_v[rb, m, pl.ds(k * LANES, LANES)] + pchunk[k]
                        p_v[pl.ds(m * PSTRIDE + k * LANES, LANES)] = v

            @plsc.parallel_loop(0, DT, unroll=4)
            def dt_body(dt):
                for di in range(DI):
                    d_splat = jnp.full((LANES,), dt * DI + di, jnp.int32)
                    for g in range(BB // LANES):
                        v = plsc.load_gather(p_v, [m65[g] + d_splat])
                        ob_v[rb, dt, di, pl.ds(g * LANES, LANES)] = v

        def p_body(p, carry):
            for rb in range(RB):
                l = p * RB + rb
                pltpu.make_async_copy(
                    tok_hbm.at[xt_v.at[l]], rows_v.at[rb], gsem[rb]
                ).wait()

                @pl.when(p > 0)
                def _():
                    pltpu.make_async_copy(
                        ob_v.at[rb], out_hbm.at[l - RB, :, j], wsem[rb]
                    ).wait()

                compute_tile(l, rb)
                pltpu.async_copy(ob_v.at[rb], out_hbm.at[l, :, j], wsem[rb])

                @pl.when(p < NP - 1)
                def _():
                    pltpu.async_copy(
                        tok_hbm.at[xt_v.at[l + RB]], rows_v.at[rb], gsem[rb]
                    )

            return carry

        lax.fori_loop(0, NP, p_body, 0)
        for rb in range(RB):
            pltpu.make_async_copy(
                ob_v.at[rb], out_hbm.at[L - RB + rb, :, j], wsem[rb]
            ).wait()

    return body


def kernel(x, token_table, pos_table):
    xi = x.reshape(-1).astype(jnp.int32)
    buf = _embed_kernel()(xi, token_table, pos_table)
    return buf.transpose(2, 4, 0, 1, 3).reshape(BATCH, L, D)
